# baseline (speedup 1.0000x reference)
import jax
import jax.numpy as jnp
from jax import lax
from jax.experimental import pallas as pl
from jax.experimental.pallas import tpu as pltpu
from jax.experimental.pallas import tpu_sc as plsc

TOKEN_ID = 31999
R = 16384
H = 2048
NT = 512
NW = 32
ROWS_W = R // NW
CHUNK = 16
NCHUNK = ROWS_W // CHUNK
NBUF = 2
L = 16


def _sc_body(emb_hbm, ids_hbm, t_hbm, w_hbm, b_hbm, out_hbm,
             buf0, buf1, ids_v, t_v, w_v, b_v,
             t_smem, prefix_smem,
             sem_in0, sem_in1, sem_out0, sem_out1):
    nc = 2
    wid = lax.axis_index("s") * nc + lax.axis_index("c")
    wbase = wid * ROWS_W
    lanes = lax.iota(jnp.int32, L)

    pltpu.sync_copy(ids_hbm, ids_v)
    pltpu.sync_copy(t_hbm, t_v)
    pltpu.sync_copy(w_hbm, w_v)
    pltpu.sync_copy(b_hbm, b_v)

    bufs = (buf0, buf1)
    sems_in = (sem_in0, sem_in1)
    sems_out = (sem_out0, sem_out1)

    def load(c, bi):
        pltpu.make_async_copy(
            emb_hbm.at[pl.ds(wbase + c * CHUNK, CHUNK)], bufs[bi], sems_in[bi]
        ).start()

    def wait_load(bi):
        pltpu.make_async_copy(
            emb_hbm.at[pl.ds(wbase, CHUNK)], bufs[bi], sems_in[bi]
        ).wait()

    def store(c, bi):
        pltpu.make_async_copy(
            bufs[bi], out_hbm.at[pl.ds(wbase + c * CHUNK, CHUNK)], sems_out[bi]
        ).start()

    def wait_store(bi):
        pltpu.make_async_copy(
            bufs[bi], out_hbm.at[pl.ds(wbase, CHUNK)], sems_out[bi]
        ).wait()

    load(0, 0)
    load(1, 1)

    for v in range(NT // L):
        tv = (t_v[pl.ds(v * L, L)] - 1175.0) * (1.0 / 2350.0)
        for l in range(L):
            t_smem[v * L + l] = jnp.sum(jnp.where(lanes == l, tv, 0.0))

    def pass_a(i, cnt):
        v = ids_v[pl.ds(i * L, L)]
        return cnt + jnp.sum((v == TOKEN_ID).astype(jnp.int32))

    run = lax.fori_loop(0, wid * NCHUNK, pass_a, jnp.int32(0))

    for v in range(NCHUNK):
        prefix_smem[v] = run
        mvec = ids_v[pl.ds(wbase + v * L, L)] == TOKEN_ID
        run = run + jnp.sum(mvec.astype(jnp.int32))

    def patch(c, bi):
        mvec = ids_v[pl.ds(wbase + c * L, L)] == TOKEN_ID
        csum = plsc.cumsum(mvec.astype(jnp.int32))
        gvec = prefix_smem[c] + csum - 1

        def cond(m):
            return jnp.any(m)

        def body(m):
            lane = jnp.max(plsc.all_reduce_ffs(m))
            g = jnp.max(jnp.where(lanes == lane, gvec, -1))
            t_s = t_smem[g]
            row_idx = jnp.full((L,), lane, jnp.int32)
            for h in range(H // L):
                vals = t_s * w_v[pl.ds(h * L, L)] + b_v[pl.ds(h * L, L)]
                plsc.store_scatter(bufs[bi], [row_idx, h * L + lanes], vals)
            return m & (lanes != lane)

        lax.while_loop(cond, body, mvec)

    def ring_step(g, carry):
        for bi in range(NBUF):
            c = g * NBUF + bi
            wait_load(bi)
            patch(c, bi)
            store(c, bi)

            @pl.when(c + NBUF < NCHUNK)
            def _reload():
                wait_store(bi)
                load(c + NBUF, bi)

        return carry

    lax.fori_loop(0, NCHUNK // NBUF, ring_step, jnp.int32(0))
    wait_store(0)
    wait_store(1)


@jax.jit
def _sc_call(emb2, ids1, t, w1, b1):
    mesh = plsc.VectorSubcoreMesh(core_axis_name="c", subcore_axis_name="s")
    kfn = pl.kernel(
        _sc_body,
        out_type=jax.ShapeDtypeStruct((R, H), jnp.float32),
        mesh=mesh,
        compiler_params=pltpu.CompilerParams(needs_layout_passes=False),
        scratch_types=[
            pltpu.VMEM((CHUNK, H), jnp.float32),
            pltpu.VMEM((CHUNK, H), jnp.float32),
            pltpu.VMEM((R,), jnp.int32),
            pltpu.VMEM((NT,), jnp.float32),
            pltpu.VMEM((H,), jnp.float32),
            pltpu.VMEM((H,), jnp.float32),
            pltpu.SMEM((NT,), jnp.float32),
            pltpu.SMEM((NCHUNK,), jnp.int32),
            pltpu.SemaphoreType.DMA,
            pltpu.SemaphoreType.DMA,
            pltpu.SemaphoreType.DMA,
            pltpu.SemaphoreType.DMA,
        ],
    )
    return kfn(emb2, ids1, t, w1, b1)


def kernel(inputs_embeds, input_ids, t_indices, W, b):
    B, S, Hd = inputs_embeds.shape
    emb2 = inputs_embeds.reshape(B * S, Hd)
    ids1 = input_ids.reshape(B * S)
    out = _sc_call(emb2, ids1, t_indices, W.reshape(Hd), b)
    return out.reshape(B, S, Hd)
